# baseline (speedup 1.0000x reference)
import functools

import jax
import jax.numpy as jnp
from jax.experimental import pallas as pl
from jax.experimental.pallas import tpu as pltpu
from jax.experimental.pallas import tpu_sc as plsc

WIN = 128


def _tc_transpose_idx(x):
    _, b, l = x.shape

    def body(x_ref, o0_ref, o1_ref):
        o0_ref[...] = x_ref[0].T
        o1_ref[...] = x_ref[1].T

    return pl.pallas_call(
        body,
        out_shape=(jax.ShapeDtypeStruct((l, b), jnp.int32),
                   jax.ShapeDtypeStruct((l, b), jnp.int32)),
    )(x)


def _sc_gather_word(word_embeds, x0):
    l, b = x0.shape
    rows = l * b
    nwin = b // WIN
    embed = word_embeds.shape[1]
    mesh = plsc.VectorSubcoreMesh(core_axis_name="core", subcore_axis_name="subcore")

    @functools.partial(
        pl.kernel,
        out_type=jax.ShapeDtypeStruct((rows, embed), jnp.float32),
        mesh=mesh,
    )
    def k(wtab_hbm, x0_hbm, wout_hbm):
        def body(i0_vmem, wo_vmem):
            pltpu.sync_copy(wtab_hbm.at[i0_vmem.at[0]], wo_vmem)

        pltpu.emit_pipeline(
            body,
            grid=(rows // WIN,),
            in_specs=[pl.BlockSpec((1, WIN), lambda i: (i // nwin, i % nwin))],
            out_specs=[pl.BlockSpec((WIN, embed), lambda i: (i, 0))],
            core_axis_name=("core", "subcore"),
            dimension_semantics=(pltpu.PARALLEL,),
        )(x0_hbm, wout_hbm)

    return k(word_embeds, x0)


def _sc_gather_dw(doc_weights, x1):
    l, b = x1.shape
    rows = l * b
    nwin = b // WIN
    topics = doc_weights.shape[1]
    mesh = plsc.VectorSubcoreMesh(core_axis_name="core", subcore_axis_name="subcore")

    @functools.partial(
        pl.kernel,
        out_type=jax.ShapeDtypeStruct((rows, 128), jnp.float32),
        mesh=mesh,
        compiler_params=pltpu.CompilerParams(use_tc_tiling_on_sc=False),
    )
    def k(dtab_hbm, x1_hbm, dout_hbm):
        def body(i1_vmem, do_vmem):
            pltpu.sync_copy(dtab_hbm.at[i1_vmem.at[0]], do_vmem)

        pltpu.emit_pipeline(
            body,
            grid=(rows // WIN,),
            in_specs=[pl.BlockSpec((1, WIN), lambda i: (i // nwin, i % nwin))],
            out_specs=[pl.BlockSpec((WIN, topics), lambda i: (i, 0))],
            core_axis_name=("core", "subcore"),
            dimension_semantics=(pltpu.PARALLEL,),
        )(x1_hbm, dout_hbm)

    return k(doc_weights, x1)


def _tc_finish_chunk(dw3c, word3, te_t, prev_out, chunk, nchunks, bs):
    l, bc, _ = dw3c.shape
    topics = te_t.shape[0]
    embed = word3.shape[2]
    b = bc * nchunks
    nbc = bc // bs
    base = chunk * nbc

    def body(dw_ref, w_ref, te_ref, *rest):
        o_ref = rest[-1]
        e = jnp.exp(dw_ref[:, :, :topics])
        s = jnp.sum(e, axis=0, keepdims=True)
        p = (e / s).reshape(l * bs, topics)
        doc = jnp.dot(p, te_ref[...], preferred_element_type=jnp.float32)
        o_ref[...] = (doc + w_ref[...].reshape(l * bs, embed)).reshape(
            l, bs, embed)

    in_specs = [
        pl.BlockSpec((l, bs, 128), lambda i: (0, i, 0)),
        pl.BlockSpec((l, bs, embed), lambda i: (0, i, 0)),
        pl.BlockSpec((topics, embed), lambda i: (0, 0)),
    ]
    args = [dw3c, word3, te_t]
    aliases = {}
    if prev_out is not None:
        in_specs.append(pl.BlockSpec(memory_space=pl.ANY))
        args.append(prev_out)
        aliases = {3: 0}

    return pl.pallas_call(
        body,
        grid=(nbc,),
        in_specs=in_specs,
        out_specs=pl.BlockSpec((l, bs, embed), lambda i: (0, base + i, 0)),
        out_shape=jax.ShapeDtypeStruct((l, b, embed), jnp.float32),
        input_output_aliases=aliases,
    )(*args)


def kernel(x, word_embeds, doc_weights, topic_embeds):
    _, b, l = x.shape
    embed = word_embeds.shape[1]
    rows = b * l

    x0, x1 = _tc_transpose_idx(x)
    te_t = topic_embeds.T

    nchunks = 2
    bc = b // nchunks
    out3 = None
    for c in range(nchunks):
        x0c = jax.lax.slice(x0, (0, c * bc), (l, (c + 1) * bc))
        x1c = jax.lax.slice(x1, (0, c * bc), (l, (c + 1) * bc))
        word_gc = _sc_gather_word(word_embeds, x0c)
        dw_gc = _sc_gather_dw(doc_weights, x1c)
        word3c = word_gc.reshape(l, bc, embed)
        dw3c = dw_gc.reshape(l, bc, 128)
        out3 = _tc_finish_chunk(dw3c, word3c, te_t, out3, c, nchunks, 256)
    return jnp.transpose(out3, (1, 0, 2))


# speedup vs baseline: 1.0814x; 1.0814x over previous
import functools

import jax
import jax.numpy as jnp
from jax.experimental import pallas as pl
from jax.experimental.pallas import tpu as pltpu
from jax.experimental.pallas import tpu_sc as plsc

WIN = 128


def _tc_transpose_idx(x):
    _, b, l = x.shape

    def body(x_ref, o0_ref, o1_ref):
        o0_ref[...] = x_ref[0].T
        o1_ref[...] = x_ref[1].T

    return pl.pallas_call(
        body,
        out_shape=(jax.ShapeDtypeStruct((l, b), jnp.int32),
                   jax.ShapeDtypeStruct((l, b), jnp.int32)),
    )(x)


def _sc_gather_word(word_embeds, x0):
    l, b = x0.shape
    rows = l * b
    nwin = b // WIN
    embed = word_embeds.shape[1]
    mesh = plsc.VectorSubcoreMesh(core_axis_name="core", subcore_axis_name="subcore")

    @functools.partial(
        pl.kernel,
        out_type=jax.ShapeDtypeStruct((rows, embed), jnp.float32),
        mesh=mesh,
    )
    def k(wtab_hbm, x0_hbm, wout_hbm):
        def body(i0_vmem, wo_vmem):
            pltpu.sync_copy(wtab_hbm.at[i0_vmem.at[0]], wo_vmem)

        pltpu.emit_pipeline(
            body,
            grid=(rows // WIN,),
            in_specs=[pl.BlockSpec((1, WIN), lambda i: (i // nwin, i % nwin))],
            out_specs=[pl.BlockSpec((WIN, embed), lambda i: (i, 0))],
            core_axis_name=("core", "subcore"),
            dimension_semantics=(pltpu.PARALLEL,),
        )(x0_hbm, wout_hbm)

    return k(word_embeds, x0)


def _sc_gather_dw(doc_weights, x1):
    l, b = x1.shape
    rows = l * b
    nwin = b // WIN
    topics = doc_weights.shape[1]
    mesh = plsc.VectorSubcoreMesh(core_axis_name="core", subcore_axis_name="subcore")

    @functools.partial(
        pl.kernel,
        out_type=jax.ShapeDtypeStruct((rows, 128), jnp.float32),
        mesh=mesh,
        compiler_params=pltpu.CompilerParams(use_tc_tiling_on_sc=False),
    )
    def k(dtab_hbm, x1_hbm, dout_hbm):
        def body(i1_vmem, do_vmem):
            pltpu.sync_copy(dtab_hbm.at[i1_vmem.at[0]], do_vmem)

        pltpu.emit_pipeline(
            body,
            grid=(rows // WIN,),
            in_specs=[pl.BlockSpec((1, WIN), lambda i: (i // nwin, i % nwin))],
            out_specs=[pl.BlockSpec((WIN, topics), lambda i: (i, 0))],
            core_axis_name=("core", "subcore"),
            dimension_semantics=(pltpu.PARALLEL,),
        )(x1_hbm, dout_hbm)

    return k(doc_weights, x1)


def _tc_finish_chunk(dw3c, word3, te_t, prev_out, chunk, nchunks, bs):
    l, bc, _ = dw3c.shape
    topics = te_t.shape[0]
    embed = word3.shape[2]
    b = bc * nchunks
    nbc = bc // bs
    base = chunk * nbc

    def body(dw_ref, w_ref, te_ref, *rest):
        o_ref = rest[-1]
        e = jnp.exp(dw_ref[:, :, :topics])
        s = jnp.sum(e, axis=0, keepdims=True)
        p = (e / s).reshape(l * bs, topics)
        doc = jnp.dot(p, te_ref[...], preferred_element_type=jnp.float32)
        o_ref[...] = (doc + w_ref[...].reshape(l * bs, embed)).reshape(
            l, bs, embed)

    in_specs = [
        pl.BlockSpec((l, bs, 128), lambda i: (0, i, 0)),
        pl.BlockSpec((l, bs, embed), lambda i: (0, base + i, 0)),
        pl.BlockSpec((topics, embed), lambda i: (0, 0)),
    ]
    args = [dw3c, word3, te_t]
    aliases = {}
    if prev_out is not None:
        in_specs.append(pl.BlockSpec(memory_space=pl.ANY))
        args.append(prev_out)
        aliases = {3: 0}

    return pl.pallas_call(
        body,
        grid=(nbc,),
        in_specs=in_specs,
        out_specs=pl.BlockSpec((l, bs, embed), lambda i: (0, base + i, 0)),
        out_shape=jax.ShapeDtypeStruct((l, b, embed), jnp.float32),
        input_output_aliases=aliases,
    )(*args)


def kernel(x, word_embeds, doc_weights, topic_embeds):
    _, b, l = x.shape
    embed = word_embeds.shape[1]
    rows = b * l

    x0, x1 = _tc_transpose_idx(x)
    te_t = topic_embeds.T
    word_g = _sc_gather_word(word_embeds, x0)
    word3 = word_g.reshape(l, b, embed)

    nchunks = 2
    bc = b // nchunks
    out3 = None
    for c in range(nchunks):
        x1c = jax.lax.slice(x1, (0, c * bc), (l, (c + 1) * bc))
        dw_gc = _sc_gather_dw(doc_weights, x1c)
        dw3c = dw_gc.reshape(l, bc, 128)
        out3 = _tc_finish_chunk(dw3c, word3, te_t, out3, c, nchunks, 256)
    return jnp.transpose(out3, (1, 0, 2))


# speedup vs baseline: 1.0944x; 1.0120x over previous
import functools

import jax
import jax.numpy as jnp
from jax.experimental import pallas as pl
from jax.experimental.pallas import tpu as pltpu
from jax.experimental.pallas import tpu_sc as plsc

WIN = 128


def _tc_transpose_idx(x):
    _, b, l = x.shape

    def body(x_ref, o0_ref, o1_ref):
        o0_ref[...] = x_ref[0].T
        o1_ref[...] = x_ref[1].T

    return pl.pallas_call(
        body,
        out_shape=(jax.ShapeDtypeStruct((l, b), jnp.int32),
                   jax.ShapeDtypeStruct((l, b), jnp.int32)),
    )(x)


def _sc_gather_word(word_embeds, x0):
    l, b = x0.shape
    rows = l * b
    nwin = b // WIN
    embed = word_embeds.shape[1]
    mesh = plsc.VectorSubcoreMesh(core_axis_name="core", subcore_axis_name="subcore")

    @functools.partial(
        pl.kernel,
        out_type=jax.ShapeDtypeStruct((rows, embed), jnp.float32),
        mesh=mesh,
    )
    def k(wtab_hbm, x0_hbm, wout_hbm):
        def body(i0_vmem, wo_vmem):
            pltpu.sync_copy(wtab_hbm.at[i0_vmem.at[0]], wo_vmem)

        pltpu.emit_pipeline(
            body,
            grid=(rows // WIN,),
            in_specs=[pl.BlockSpec((1, WIN), lambda i: (i // nwin, i % nwin))],
            out_specs=[pl.BlockSpec((WIN, embed), lambda i: (i, 0))],
            core_axis_name=("core", "subcore"),
            dimension_semantics=(pltpu.PARALLEL,),
        )(x0_hbm, wout_hbm)

    return k(word_embeds, x0)


def _sc_gather_dw(doc_weights, x1):
    l, b = x1.shape
    rows = l * b
    nwin = b // WIN
    topics = doc_weights.shape[1]
    mesh = plsc.VectorSubcoreMesh(core_axis_name="core", subcore_axis_name="subcore")

    @functools.partial(
        pl.kernel,
        out_type=jax.ShapeDtypeStruct((rows, 128), jnp.float32),
        mesh=mesh,
        compiler_params=pltpu.CompilerParams(use_tc_tiling_on_sc=False),
    )
    def k(dtab_hbm, x1_hbm, dout_hbm):
        def body(i1_vmem, do_vmem):
            pltpu.sync_copy(dtab_hbm.at[i1_vmem.at[0]], do_vmem)

        pltpu.emit_pipeline(
            body,
            grid=(rows // WIN,),
            in_specs=[pl.BlockSpec((1, WIN), lambda i: (i // nwin, i % nwin))],
            out_specs=[pl.BlockSpec((WIN, topics), lambda i: (i, 0))],
            core_axis_name=("core", "subcore"),
            dimension_semantics=(pltpu.PARALLEL,),
        )(x1_hbm, dout_hbm)

    return k(doc_weights, x1)


def _tc_finish_chunk(dw3c, word3, te_t, prev_out, chunk, nchunks, bs):
    l, bc, _ = dw3c.shape
    topics = te_t.shape[0]
    embed = word3.shape[2]
    b = bc * nchunks
    nbc = bc // bs
    base = chunk * nbc

    def body(dw_ref, w_ref, te_ref, *rest):
        o_ref = rest[-1]
        e = jnp.exp(dw_ref[:, :, :topics])
        s = jnp.sum(e, axis=0, keepdims=True)
        p = (e / s).reshape(l * bs, topics)
        doc = jnp.dot(p, te_ref[...], preferred_element_type=jnp.float32)
        o_ref[...] = (doc + w_ref[...].reshape(l * bs, embed)).reshape(
            l, bs, embed)

    in_specs = [
        pl.BlockSpec((l, bs, 128), lambda i: (0, i, 0)),
        pl.BlockSpec((l, bs, embed), lambda i: (0, base + i, 0)),
        pl.BlockSpec((topics, embed), lambda i: (0, 0)),
    ]
    args = [dw3c, word3, te_t]
    aliases = {}
    if prev_out is not None:
        in_specs.append(pl.BlockSpec(memory_space=pl.ANY))
        args.append(prev_out)
        aliases = {3: 0}

    return pl.pallas_call(
        body,
        grid=(nbc,),
        in_specs=in_specs,
        out_specs=pl.BlockSpec((l, bs, embed), lambda i: (0, base + i, 0)),
        out_shape=jax.ShapeDtypeStruct((l, b, embed), jnp.float32),
        input_output_aliases=aliases,
    )(*args)


def kernel(x, word_embeds, doc_weights, topic_embeds):
    _, b, l = x.shape
    embed = word_embeds.shape[1]
    rows = b * l

    x0, x1 = _tc_transpose_idx(x)
    te_t = topic_embeds.T
    word_g = _sc_gather_word(word_embeds, x0)
    word3 = word_g.reshape(l, b, embed)

    nchunks = 2
    bc = b // nchunks
    out3 = None
    for c in range(nchunks):
        x1c = jax.lax.slice(x1, (0, c * bc), (l, (c + 1) * bc))
        dw_gc = _sc_gather_dw(doc_weights, x1c)
        dw3c = dw_gc.reshape(l, bc, 128)
        out3 = _tc_finish_chunk(dw3c, word3, te_t, out3, c, nchunks, 512)
    return jnp.transpose(out3, (1, 0, 2))
